# baseline (speedup 1.0000x reference)
import functools

import jax
import jax.numpy as jnp
from jax import lax
from jax.experimental import pallas as pl
from jax.experimental.pallas import tpu as pltpu
from jax.experimental.pallas import tpu_sc as plsc

NUM_USERS = 50000
NUM_ITEMS = 50000
N_NODES = NUM_USERS + NUM_ITEMS
D = 32
E = 1600000

NC = 2
NS = 16
L = 16
NW = NC * NS

HALF = N_NODES // NC
TRASH = HALF
ACC_ROWS = 51200
ZROWS = ACC_ROWS // NS
WBC = 80

XROWS = N_NODES + 8
PAD_ID = N_NODES
CHUNK = 128
BLK = 256
NSEG = BLK // CHUNK

SPU = E // NW
CAP = 100608
NREG = NW * NC

PCH = 1024
NCHF = SPU // PCH
TAIL = SPU - NCHF * PCH
FL = 1024
DUMP = FL + 272
STG = FL + 288

_mesh = plsc.VectorSubcoreMesh(
    core_axis_name="c", subcore_axis_name="s", num_cores=NC, num_subcores=NS
)


@functools.partial(
    pl.kernel,
    out_type=[
        jax.ShapeDtypeStruct((NREG * CAP,), jnp.int32),
        jax.ShapeDtypeStruct((NREG * CAP,), jnp.int32),
        jax.ShapeDtypeStruct((NREG * L,), jnp.int32),
    ],
    mesh=_mesh,
    scratch_types=[
        pltpu.VMEM((PCH,), jnp.int32),
        pltpu.VMEM((PCH,), jnp.int32),
        pltpu.VMEM((PCH,), jnp.int32),
        pltpu.VMEM((PCH,), jnp.int32),
        pltpu.VMEM((STG,), jnp.int32),
        pltpu.VMEM((STG,), jnp.int32),
        pltpu.VMEM((STG,), jnp.int32),
        pltpu.VMEM((STG,), jnp.int32),
        pltpu.VMEM((L,), jnp.int32),
        pltpu.SemaphoreType.DMA,
    ],
    compiler_params=pltpu.CompilerParams(use_tc_tiling_on_sc=False,
                                         needs_layout_passes=False),
)
def _partition(row_hbm, col_hbm, psrc, pdst, cnt,
               iis0, iid0, iis1, iid1, bs0, bd0, bs1, bd1, cv, sem):
    c = lax.axis_index("c")
    s = lax.axis_index("s")
    w = s * NC + c
    stripe = w * SPU
    stg = ((bs0, bd0), (bs1, bd1))
    chk = ((iis0, iid0), (iis1, iid1))
    uhalf = jnp.uint32(HALF)

    def load_chunk(S, k):
        off = pl.multiple_of(stripe + lax.rem(k, NCHF) * PCH, 8)
        pltpu.async_copy(row_hbm.at[pl.ds(off, PCH)], chk[S][0], sem)
        pltpu.async_copy(col_hbm.at[pl.ds(off, PCH)], chk[S][1], sem)

    def wait_chunk(S):
        pltpu.make_async_copy(row_hbm.at[pl.ds(0, PCH)], chk[S][0], sem).wait()
        pltpu.make_async_copy(col_hbm.at[pl.ds(0, PCH)], chk[S][1], sem).wait()

    def flush_check(ns, outs):
        for tc in range(NC):
            bsx, bdx = stg[tc]
            r = (w * NC + tc) * CAP
            nf, of = ns[tc], outs[tc]

            def flush(of=of, bsx=bsx, bdx=bdx, r=r):
                ro = pl.multiple_of(r + of, 8)
                pltpu.sync_copy(bsx.at[pl.ds(0, FL)],
                                psrc.at[pl.ds(ro, FL)])
                pltpu.sync_copy(bdx.at[pl.ds(0, FL)],
                                pdst.at[pl.ds(ro, FL)])
                for i in range(16):
                    vs = bsx[pl.ds(FL + i * L, L)]
                    vd = bdx[pl.ds(FL + i * L, L)]
                    bsx[pl.ds(i * L, L)] = vs
                    bdx[pl.ds(i * L, L)] = vd

            do = lax.reduce_max(nf, (0,)) >= FL
            pl.when(do)(flush)
            ns[tc] = jnp.where(do, nf - FL, nf)
            outs[tc] = jnp.where(do, of + FL, of)

    def classify(rbuf, cbuf, j, ns, outs):
        rv = rbuf[pl.ds(j * L, L)]
        cvv = cbuf[pl.ds(j * L, L)]
        for dv, sv in ((rv, cvv), (cvv, rv)):
            for tc in range(NC):
                bsx, bdx = stg[tc]
                du = plsc.bitcast(dv - tc * HALF, jnp.uint32)
                m = du < uhalf
                cs = plsc.cumsum(m.astype(jnp.int32))
                pos = jnp.where(m, ns[tc] + cs - 1, DUMP)
                plsc.store_scatter(bsx, [pos], sv)
                plsc.store_scatter(bdx, [pos], plsc.bitcast(du, jnp.int32))
                ns[tc] = ns[tc] + plsc.all_reduce_population_count(m)
        if j % 8 == 7:
            flush_check(ns, outs)

    def process(S, carry, k):
        n0, o0, n1, o1 = carry
        rbuf, cbuf = chk[S]
        wait_chunk(S)
        load_chunk(1 - S, k + 1)
        ns = [n0, n1]
        outs = [o0, o1]
        for j in range(PCH // L):
            classify(rbuf, cbuf, j, ns, outs)
        return (ns[0], outs[0], ns[1], outs[1])

    load_chunk(0, 0)

    def pair(p, carry):
        carry = process(0, carry, 2 * p)
        carry = process(1, carry, 2 * p + 1)
        return carry

    zv = jnp.zeros((L,), jnp.int32)
    n0, o0, n1, o1 = lax.fori_loop(0, NCHF // 2, pair,
                                   (zv, jnp.int32(0), zv, jnp.int32(0)))
    wait_chunk(0)
    toff = pl.multiple_of(stripe + NCHF * PCH, 8)
    pltpu.sync_copy(row_hbm.at[pl.ds(toff, TAIL)], iis0.at[pl.ds(0, TAIL)])
    pltpu.sync_copy(col_hbm.at[pl.ds(toff, TAIL)], iid0.at[pl.ds(0, TAIL)])
    ns = [n0, n1]
    outs = [o0, o1]
    for j in range(TAIL // L):
        classify(iis0, iid0, j, ns, outs)
    n0, o0, n1, o1 = ns[0], outs[0], ns[1], outs[1]

    fins = ((n0, o0), (n1, o1))
    pad_src = jnp.broadcast_to(jnp.int32(PAD_ID), (L,))
    pad_dst = jnp.broadcast_to(jnp.int32(TRASH), (L,))
    for tc in range(NC):
        bsx, bdx = stg[tc]
        r = (w * NC + tc) * CAP
        nv, out = fins[tc]
        n = lax.reduce_max(nv, (0,))

        def flush256(k, out, bsx=bsx, bdx=bdx, r=r):
            ko = pl.multiple_of(k * 256, 8)
            ro = pl.multiple_of(r + out, 8)
            pltpu.sync_copy(bsx.at[pl.ds(ko, 256)],
                            psrc.at[pl.ds(ro, 256)])
            pltpu.sync_copy(bdx.at[pl.ds(ko, 256)],
                            pdst.at[pl.ds(ro, 256)])
            return out + 256

        q = n // 256
        out = lax.fori_loop(0, q, flush256, out)
        rem = n - q * 256
        for i in range(16):
            qo = pl.multiple_of(q * 256 + i * L, 8)
            vs = bsx[pl.ds(qo, L)]
            vd = bdx[pl.ds(qo, L)]
            bsx[pl.ds(i * L, L)] = vs
            bdx[pl.ds(i * L, L)] = vd
        total = out + rem
        target = lax.max(jnp.int32(512), ((total + 511) // 512) * 512)
        npad = target - total

        def padv(k, _, bsx=bsx, bdx=bdx, rem=rem):
            pos = rem + k * L + lax.iota(jnp.int32, L)
            plsc.store_scatter(bsx, [pos], pad_src)
            plsc.store_scatter(bdx, [pos], pad_dst)
            return _

        lax.fori_loop(0, (npad + L - 1) // L, padv, 0)
        out = lax.fori_loop(0, (rem + npad) // 256, flush256, out)
        cv[pl.ds(0, L)] = jnp.broadcast_to(out, (L,))
        co = pl.multiple_of((w * NC + tc) * L, 8)
        pltpu.sync_copy(cv, cnt.at[pl.ds(co, L)])


def _make_layer(scale, out_rows):
    @functools.partial(
        pl.kernel,
        out_type=jax.ShapeDtypeStruct((out_rows, D), jnp.float32),
        mesh=_mesh,
        scratch_types=[
            pltpu.VMEM_SHARED((ACC_ROWS, D), jnp.float32),
            pltpu.VMEM((BLK,), jnp.int32),
            pltpu.VMEM((BLK,), jnp.int32),
            pltpu.VMEM((BLK,), jnp.int32),
            pltpu.VMEM((BLK,), jnp.int32),
            pltpu.VMEM((NSEG, CHUNK), jnp.int32),
            pltpu.VMEM((NSEG, CHUNK), jnp.int32),
            pltpu.VMEM((BLK, D), jnp.float32),
            pltpu.VMEM((BLK, D), jnp.float32),
            pltpu.VMEM((CHUNK, D), jnp.float32),
            pltpu.VMEM((L,), jnp.int32),
            pltpu.SemaphoreType.DMA,
            pltpu.SemaphoreType.DMA,
            pltpu.SemaphoreType.DMA,
            pltpu.SemaphoreType.DMA,
        ],
        compiler_params=pltpu.CompilerParams(use_tc_tiling_on_sc=False,
                                             needs_layout_passes=False),
    )
    def layer(x_hbm, psrc, pdst, cnt, y_hbm,
              acc, is0, is1, il0, il1, ld0, ld1, xs0, xs1, sb, cv,
              sem_i, sem_g, sem_s0, sem_s1):
        sem_s = (sem_s0, sem_s1)
        c = lax.axis_index("c")
        s = lax.axis_index("s")
        base = c * HALF
        sets = ((is0, il0, ld0, xs0), (is1, il1, ld1, xs1))

        zero = jnp.zeros((L,), jnp.float32)

        def zfill(i, carry):
            sb[i, pl.ds(0, L)] = zero
            sb[i, pl.ds(L, L)] = zero
            return carry

        lax.fori_loop(0, CHUNK, zfill, 0)
        for k in range(ZROWS // CHUNK):
            pltpu.sync_copy(sb, acc.at[pl.ds(s * ZROWS + k * CHUNK, CHUNK)])
        plsc.subcore_barrier()

        def do_region(r):
            rbase = r * CAP
            pltpu.sync_copy(cnt.at[pl.ds(pl.multiple_of(r * L, 8), L)], cv)
            total = lax.reduce_max(cv[pl.ds(0, L)], (0,))
            nb = total // BLK

            def load_idx(S, blk):
                si, li = sets[S][0], sets[S][1]
                off = pl.multiple_of(rbase + blk * BLK, 8)
                pltpu.async_copy(psrc.at[pl.ds(off, BLK)], si, sem_i)
                pltpu.async_copy(pdst.at[pl.ds(off, BLK)], li, sem_i)

            def wait_idx(S):
                si, li = sets[S][0], sets[S][1]
                pltpu.make_async_copy(psrc.at[pl.ds(0, BLK)], si, sem_i).wait()
                pltpu.make_async_copy(pdst.at[pl.ds(0, BLK)], li, sem_i).wait()

            def copy2d(S):
                li, ld = sets[S][1], sets[S][2]
                for j in range(BLK // L):
                    seg, lane = j // (CHUNK // L), (j % (CHUNK // L)) * L
                    ld[seg, pl.ds(lane, L)] = li[pl.ds(j * L, L)]

            def fire_gathers(S):
                si, xs = sets[S][0], sets[S][3]
                for j in range(NSEG):
                    sl = pl.ds(j * CHUNK, CHUNK)
                    pltpu.async_copy(x_hbm.at[si.at[sl]], xs.at[sl], sem_g)

            def drain_gathers(S):
                si, xs = sets[S][0], sets[S][3]
                for j in range(NSEG):
                    sl = pl.ds(j * CHUNK, CHUNK)
                    pltpu.make_async_copy(x_hbm.at[si.at[sl]], xs.at[sl],
                                          sem_g).wait()

            def fire_scatters(S):
                ld, xs = sets[S][2], sets[S][3]
                for j in range(NSEG):
                    sl = pl.ds(j * CHUNK, CHUNK)
                    pltpu.async_copy(xs.at[sl], acc.at[ld.at[j]], sem_s[S],
                                     add=True)

            def drain_scatters(S):
                ld, xs = sets[S][2], sets[S][3]
                for j in range(NSEG):
                    sl = pl.ds(j * CHUNK, CHUNK)
                    pltpu.make_async_copy(xs.at[sl], acc.at[ld.at[j]],
                                          sem_s[S]).wait()

            load_idx(0, 0)
            wait_idx(0)
            copy2d(0)
            fire_gathers(0)
            load_idx(1, 1)
            for j in range(BLK // L):
                seg, lane = j // (CHUNK // L), (j % (CHUNK // L)) * L
                ld1[seg, pl.ds(lane, L)] = jnp.broadcast_to(
                    jnp.int32(TRASH), (L,))
            for j in range(NSEG):
                pltpu.async_copy(sb.at[pl.ds(0, CHUNK)], acc.at[ld1.at[j]],
                                 sem_s[1], add=True)

            def substep(S, T, b):
                drain_gathers(S)
                wait_idx(T)
                drain_scatters(T)
                copy2d(T)
                fire_gathers(T)
                load_idx(S, lax.rem(b + 2, nb))
                fire_scatters(S)

            def pair(p, carry):
                substep(0, 1, 2 * p)
                substep(1, 0, 2 * p + 1)
                return carry

            lax.fori_loop(0, nb // 2, pair, 0)
            drain_gathers(0)
            wait_idx(1)
            drain_scatters(1)

        do_region(4 * s + c)
        do_region(4 * s + 2 + c)
        plsc.subcore_barrier()

        wstripe = s * ZROWS
        if scale is None:
            def wb(k, carry):
                r0 = pl.multiple_of(wstripe + k * WBC, 8)

                @pl.when(r0 < HALF)
                def _():
                    pltpu.sync_copy(acc.at[pl.ds(r0, WBC)],
                                    y_hbm.at[pl.ds(base + r0, WBC)])

                return carry

            lax.fori_loop(0, ZROWS // WBC, wb, 0)
        else:
            def wb(k, carry):
                r0 = pl.multiple_of(wstripe + k * WBC, 8)

                @pl.when(r0 < HALF)
                def _():
                    pltpu.sync_copy(acc.at[pl.ds(r0, WBC)], sb.at[pl.ds(0, WBC)])

                    def scl(i, cc):
                        sb[i, pl.ds(0, L)] = sb[i, pl.ds(0, L)] * scale
                        sb[i, pl.ds(L, L)] = sb[i, pl.ds(L, L)] * scale
                        return cc

                    lax.fori_loop(0, WBC, scl, 0)
                    pltpu.sync_copy(sb.at[pl.ds(0, WBC)],
                                    y_hbm.at[pl.ds(base + r0, WBC)])

                return carry

            lax.fori_loop(0, ZROWS // WBC, wb, 0)

    return layer


_layer_raw = _make_layer(None, XROWS)
_layer_out = _make_layer(0.25, N_NODES)


def kernel(edge_index, user_embedding, item_embedding):
    ei = edge_index.astype(jnp.int32)
    xpad = jnp.zeros((XROWS - N_NODES, D), jnp.float32)
    x0 = jnp.concatenate([user_embedding, item_embedding, xpad], axis=0)
    psrc, pdst, cnt = _partition(ei[0], ei[1])
    x1 = _layer_raw(x0, psrc, pdst, cnt)
    return _layer_out(x1, psrc, pdst, cnt)


# speedup vs baseline: 1.0061x; 1.0061x over previous
import functools

import jax
import jax.numpy as jnp
from jax import lax
from jax.experimental import pallas as pl
from jax.experimental.pallas import tpu as pltpu
from jax.experimental.pallas import tpu_sc as plsc

NUM_USERS = 50000
NUM_ITEMS = 50000
N_NODES = NUM_USERS + NUM_ITEMS
D = 32
E = 1600000

NC = 2
NS = 16
L = 16
NW = NC * NS

HALF = N_NODES // NC
TRASH = HALF
ACC_ROWS = 51200
ZROWS = ACC_ROWS // NS
WBC = 80

XROWS = N_NODES + 8
PAD_ID = N_NODES
CHUNK = 128
BLK = 256
NSEG = BLK // CHUNK

SPU = E // NW
CAP = 100608
NREG = NW * NC

PCH = 1024
NCHF = SPU // PCH
TAIL = SPU - NCHF * PCH
FL = 1024
DUMP = FL + 272
STG = FL + 288

_mesh = plsc.VectorSubcoreMesh(
    core_axis_name="c", subcore_axis_name="s", num_cores=NC, num_subcores=NS
)


@functools.partial(
    pl.kernel,
    out_type=[
        jax.ShapeDtypeStruct((NREG * CAP,), jnp.int32),
        jax.ShapeDtypeStruct((NREG * CAP,), jnp.int32),
        jax.ShapeDtypeStruct((NREG * L,), jnp.int32),
    ],
    mesh=_mesh,
    scratch_types=[
        pltpu.VMEM((PCH,), jnp.int32),
        pltpu.VMEM((PCH,), jnp.int32),
        pltpu.VMEM((PCH,), jnp.int32),
        pltpu.VMEM((PCH,), jnp.int32),
        pltpu.VMEM((STG,), jnp.int32),
        pltpu.VMEM((STG,), jnp.int32),
        pltpu.VMEM((STG,), jnp.int32),
        pltpu.VMEM((STG,), jnp.int32),
        pltpu.VMEM((L,), jnp.int32),
        pltpu.SemaphoreType.DMA,
    ],
    compiler_params=pltpu.CompilerParams(use_tc_tiling_on_sc=False,
                                         needs_layout_passes=False),
)
def _partition(row_hbm, col_hbm, psrc, pdst, cnt,
               iis0, iid0, iis1, iid1, bs0, bd0, bs1, bd1, cv, sem):
    c = lax.axis_index("c")
    s = lax.axis_index("s")
    w = s * NC + c
    stripe = w * SPU
    stg = ((bs0, bd0), (bs1, bd1))
    chk = ((iis0, iid0), (iis1, iid1))
    uhalf = jnp.uint32(HALF)

    def load_chunk(S, k):
        off = pl.multiple_of(stripe + lax.rem(k, NCHF) * PCH, 8)
        pltpu.async_copy(row_hbm.at[pl.ds(off, PCH)], chk[S][0], sem)
        pltpu.async_copy(col_hbm.at[pl.ds(off, PCH)], chk[S][1], sem)

    def wait_chunk(S):
        pltpu.make_async_copy(row_hbm.at[pl.ds(0, PCH)], chk[S][0], sem).wait()
        pltpu.make_async_copy(col_hbm.at[pl.ds(0, PCH)], chk[S][1], sem).wait()

    def flush_check(ns, outs):
        for tc in range(NC):
            bsx, bdx = stg[tc]
            r = (w * NC + tc) * CAP
            nf, of = ns[tc], outs[tc]

            def flush(of=of, bsx=bsx, bdx=bdx, r=r):
                ro = pl.multiple_of(r + of, 8)
                pltpu.sync_copy(bsx.at[pl.ds(0, FL)],
                                psrc.at[pl.ds(ro, FL)])
                pltpu.sync_copy(bdx.at[pl.ds(0, FL)],
                                pdst.at[pl.ds(ro, FL)])
                for i in range(16):
                    vs = bsx[pl.ds(FL + i * L, L)]
                    vd = bdx[pl.ds(FL + i * L, L)]
                    bsx[pl.ds(i * L, L)] = vs
                    bdx[pl.ds(i * L, L)] = vd

            do = lax.reduce_max(nf, (0,)) >= FL
            pl.when(do)(flush)
            ns[tc] = jnp.where(do, nf - FL, nf)
            outs[tc] = jnp.where(do, of + FL, of)

    def classify(rbuf, cbuf, j, ns, outs):
        rv = rbuf[pl.ds(j * L, L)]
        cvv = cbuf[pl.ds(j * L, L)]
        for dv, sv in ((rv, cvv), (cvv, rv)):
            for tc in range(NC):
                bsx, bdx = stg[tc]
                du = plsc.bitcast(dv - tc * HALF, jnp.uint32)
                m = du < uhalf
                cs = plsc.cumsum(m.astype(jnp.int32))
                pos = jnp.where(m, ns[tc] + cs - 1, DUMP)
                plsc.store_scatter(bsx, [pos], sv)
                plsc.store_scatter(bdx, [pos], plsc.bitcast(du, jnp.int32))
                ns[tc] = ns[tc] + plsc.all_reduce_population_count(m)
        if j % 8 == 7:
            flush_check(ns, outs)

    def process(S, carry, k):
        n0, o0, n1, o1 = carry
        rbuf, cbuf = chk[S]
        wait_chunk(S)
        load_chunk(1 - S, k + 1)
        ns = [n0, n1]
        outs = [o0, o1]
        for j in range(PCH // L):
            classify(rbuf, cbuf, j, ns, outs)
        return (ns[0], outs[0], ns[1], outs[1])

    load_chunk(0, 0)

    def pair(p, carry):
        carry = process(0, carry, 2 * p)
        carry = process(1, carry, 2 * p + 1)
        return carry

    zv = jnp.zeros((L,), jnp.int32)
    n0, o0, n1, o1 = lax.fori_loop(0, NCHF // 2, pair,
                                   (zv, jnp.int32(0), zv, jnp.int32(0)))
    wait_chunk(0)
    toff = pl.multiple_of(stripe + NCHF * PCH, 8)
    pltpu.sync_copy(row_hbm.at[pl.ds(toff, TAIL)], iis0.at[pl.ds(0, TAIL)])
    pltpu.sync_copy(col_hbm.at[pl.ds(toff, TAIL)], iid0.at[pl.ds(0, TAIL)])
    ns = [n0, n1]
    outs = [o0, o1]
    for j in range(TAIL // L):
        classify(iis0, iid0, j, ns, outs)
    n0, o0, n1, o1 = ns[0], outs[0], ns[1], outs[1]

    fins = ((n0, o0), (n1, o1))
    pad_src = jnp.broadcast_to(jnp.int32(PAD_ID), (L,))
    pad_dst = jnp.broadcast_to(jnp.int32(TRASH), (L,))
    for tc in range(NC):
        bsx, bdx = stg[tc]
        r = (w * NC + tc) * CAP
        nv, out = fins[tc]
        n = lax.reduce_max(nv, (0,))

        def flush256(k, out, bsx=bsx, bdx=bdx, r=r):
            ko = pl.multiple_of(k * 256, 8)
            ro = pl.multiple_of(r + out, 8)
            pltpu.sync_copy(bsx.at[pl.ds(ko, 256)],
                            psrc.at[pl.ds(ro, 256)])
            pltpu.sync_copy(bdx.at[pl.ds(ko, 256)],
                            pdst.at[pl.ds(ro, 256)])
            return out + 256

        q = n // 256
        out = lax.fori_loop(0, q, flush256, out)
        rem = n - q * 256
        for i in range(16):
            qo = pl.multiple_of(q * 256 + i * L, 8)
            vs = bsx[pl.ds(qo, L)]
            vd = bdx[pl.ds(qo, L)]
            bsx[pl.ds(i * L, L)] = vs
            bdx[pl.ds(i * L, L)] = vd
        total = out + rem
        target = lax.max(jnp.int32(512), ((total + 511) // 512) * 512)
        npad = target - total

        def padv(k, _, bsx=bsx, bdx=bdx, rem=rem):
            pos = rem + k * L + lax.iota(jnp.int32, L)
            plsc.store_scatter(bsx, [pos], pad_src)
            plsc.store_scatter(bdx, [pos], pad_dst)
            return _

        lax.fori_loop(0, (npad + L - 1) // L, padv, 0)
        out = lax.fori_loop(0, (rem + npad) // 256, flush256, out)
        cv[pl.ds(0, L)] = jnp.broadcast_to(out, (L,))
        co = pl.multiple_of((w * NC + tc) * L, 8)
        pltpu.sync_copy(cv, cnt.at[pl.ds(co, L)])


def _make_layer(scale, out_rows):
    @functools.partial(
        pl.kernel,
        out_type=jax.ShapeDtypeStruct((out_rows, D), jnp.float32),
        mesh=_mesh,
        scratch_types=[
            pltpu.VMEM_SHARED((ACC_ROWS, D), jnp.float32),
            pltpu.VMEM((BLK,), jnp.int32),
            pltpu.VMEM((BLK,), jnp.int32),
            pltpu.VMEM((BLK,), jnp.int32),
            pltpu.VMEM((BLK,), jnp.int32),
            pltpu.VMEM((NSEG, CHUNK), jnp.int32),
            pltpu.VMEM((NSEG, CHUNK), jnp.int32),
            pltpu.VMEM((BLK, D), jnp.float32),
            pltpu.VMEM((BLK, D), jnp.float32),
            pltpu.VMEM((CHUNK, D), jnp.float32),
            pltpu.VMEM((L,), jnp.int32),
            pltpu.SemaphoreType.DMA,
            pltpu.SemaphoreType.DMA,
        ],
        compiler_params=pltpu.CompilerParams(use_tc_tiling_on_sc=False,
                                             needs_layout_passes=False),
    )
    def layer(x_hbm, psrc, pdst, cnt, y_hbm,
              acc, is0, is1, il0, il1, ld0, ld1, xs0, xs1, sb, cv,
              sem_i, sem_g):
        c = lax.axis_index("c")
        s = lax.axis_index("s")
        base = c * HALF
        sets = ((is0, il0, ld0, xs0), (is1, il1, ld1, xs1))

        zero = jnp.zeros((L,), jnp.float32)

        def zfill(i, carry):
            sb[i, pl.ds(0, L)] = zero
            sb[i, pl.ds(L, L)] = zero
            return carry

        lax.fori_loop(0, CHUNK, zfill, 0)
        for k in range(ZROWS // CHUNK):
            pltpu.sync_copy(sb, acc.at[pl.ds(s * ZROWS + k * CHUNK, CHUNK)])
        plsc.subcore_barrier()

        def do_region(r):
            rbase = r * CAP
            pltpu.sync_copy(cnt.at[pl.ds(pl.multiple_of(r * L, 8), L)], cv)
            total = lax.reduce_max(cv[pl.ds(0, L)], (0,))
            nb = total // BLK

            def load_idx(S, blk):
                si, li = sets[S][0], sets[S][1]
                off = pl.multiple_of(rbase + blk * BLK, 8)
                pltpu.async_copy(psrc.at[pl.ds(off, BLK)], si, sem_i)
                pltpu.async_copy(pdst.at[pl.ds(off, BLK)], li, sem_i)

            def wait_idx(S):
                si, li = sets[S][0], sets[S][1]
                pltpu.make_async_copy(psrc.at[pl.ds(0, BLK)], si, sem_i).wait()
                pltpu.make_async_copy(pdst.at[pl.ds(0, BLK)], li, sem_i).wait()

            def copy2d(S):
                li, ld = sets[S][1], sets[S][2]
                for j in range(BLK // L):
                    seg, lane = j // (CHUNK // L), (j % (CHUNK // L)) * L
                    ld[seg, pl.ds(lane, L)] = li[pl.ds(j * L, L)]

            def fire_gathers(S):
                si, xs = sets[S][0], sets[S][3]
                for j in range(NSEG):
                    sl = pl.ds(j * CHUNK, CHUNK)
                    pltpu.async_copy(x_hbm.at[si.at[sl]], xs.at[sl], sem_g)

            def drain_gathers(S):
                si, xs = sets[S][0], sets[S][3]
                for j in range(NSEG):
                    sl = pl.ds(j * CHUNK, CHUNK)
                    pltpu.make_async_copy(x_hbm.at[si.at[sl]], xs.at[sl],
                                          sem_g).wait()

            def scatters(S):
                ld, xs = sets[S][2], sets[S][3]
                for j in range(NSEG):
                    sl = pl.ds(j * CHUNK, CHUNK)
                    pltpu.sync_copy(xs.at[sl], acc.at[ld.at[j]], add=True)

            load_idx(0, 0)
            wait_idx(0)
            copy2d(0)
            fire_gathers(0)
            load_idx(1, 1)

            def substep(S, T, b):
                drain_gathers(S)
                wait_idx(T)
                copy2d(T)
                fire_gathers(T)
                load_idx(S, lax.rem(b + 2, nb))
                scatters(S)

            def pair(p, carry):
                substep(0, 1, 2 * p)
                substep(1, 0, 2 * p + 1)
                return carry

            lax.fori_loop(0, nb // 2, pair, 0)
            drain_gathers(0)
            wait_idx(1)

        do_region(4 * s + c)
        do_region(4 * s + 2 + c)
        plsc.subcore_barrier()

        wstripe = s * ZROWS
        if scale is None:
            def wb(k, carry):
                r0 = pl.multiple_of(wstripe + k * WBC, 8)

                @pl.when(r0 < HALF)
                def _():
                    pltpu.sync_copy(acc.at[pl.ds(r0, WBC)],
                                    y_hbm.at[pl.ds(base + r0, WBC)])

                return carry

            lax.fori_loop(0, ZROWS // WBC, wb, 0)
        else:
            def wb(k, carry):
                r0 = pl.multiple_of(wstripe + k * WBC, 8)

                @pl.when(r0 < HALF)
                def _():
                    pltpu.sync_copy(acc.at[pl.ds(r0, WBC)], sb.at[pl.ds(0, WBC)])

                    def scl(i, cc):
                        sb[i, pl.ds(0, L)] = sb[i, pl.ds(0, L)] * scale
                        sb[i, pl.ds(L, L)] = sb[i, pl.ds(L, L)] * scale
                        return cc

                    lax.fori_loop(0, WBC, scl, 0)
                    pltpu.sync_copy(sb.at[pl.ds(0, WBC)],
                                    y_hbm.at[pl.ds(base + r0, WBC)])

                return carry

            lax.fori_loop(0, ZROWS // WBC, wb, 0)

    return layer


_layer_raw = _make_layer(None, XROWS)
_layer_out = _make_layer(0.25, N_NODES)


def kernel(edge_index, user_embedding, item_embedding):
    ei = edge_index.astype(jnp.int32)
    xpad = jnp.zeros((XROWS - N_NODES, D), jnp.float32)
    x0 = jnp.concatenate([user_embedding, item_embedding, xpad], axis=0)
    psrc, pdst, cnt = _partition(ei[0], ei[1])
    x1 = _layer_raw(x0, psrc, pdst, cnt)
    return _layer_out(x1, psrc, pdst, cnt)
